# baseline (speedup 1.0000x reference)
import functools

import jax
import jax.numpy as jnp
from jax import lax
from jax.experimental import pallas as pl
from jax.experimental.pallas import tpu as pltpu
from jax.experimental.pallas import tpu_sc as plsc

BATCH = 16384
HIST = 200
DIM = 32
ROWS = 33
N = BATCH * HIST


@functools.cache
def _build():
    info = plsc.get_sparse_core_info()
    nw = info.num_cores * info.num_subcores
    per_w = N // nw
    chunk = 1280
    nchunk = per_w // chunk

    mesh = plsc.VectorSubcoreMesh(core_axis_name="c", subcore_axis_name="s")

    @functools.partial(
        pl.kernel,
        mesh=mesh,
        compiler_params=pltpu.CompilerParams(
            use_tc_tiling_on_sc=False, needs_layout_passes=False),
        out_type=jax.ShapeDtypeStruct((N * DIM,), jnp.float32),
        scratch_types=[
            pltpu.VMEM((ROWS * DIM,), jnp.float32),
            pltpu.VMEM((chunk,), jnp.int32),
            pltpu.VMEM((chunk,), jnp.int32),
            pltpu.VMEM((chunk * DIM,), jnp.float32),
            pltpu.VMEM((chunk * DIM,), jnp.float32),
            pltpu.SemaphoreType.DMA,
            pltpu.SemaphoreType.DMA,
            pltpu.SemaphoreType.DMA,
            pltpu.SemaphoreType.DMA,
        ],
    )
    def gather_kernel(idx_hbm, table_hbm, out_hbm, table_v,
                      idx0, idx1, outv0, outv1, si0, si1, so0, so1):
        wid = lax.axis_index("s") * info.num_cores + lax.axis_index("c")
        base_w = wid * per_w
        idx_bufs = (idx0, idx1)
        out_bufs = (outv0, outv1)
        isems = (si0, si1)
        osems = (so0, so1)

        pltpu.sync_copy(table_hbm, table_v)
        iota32 = lax.iota(jnp.int32, 16) * DIM

        def idx_src(g):
            return idx_hbm.at[pl.ds(base_w + g * chunk, chunk)]

        def out_dst(g):
            return out_hbm.at[pl.ds((base_w + g * chunk) * DIM, chunk * DIM)]

        def compute(idx_buf, out_buf):
            @plsc.parallel_loop(0, chunk // 16, 1, unroll=2)
            def _(i):
                idx16 = idx_buf[pl.ds(i * 16, 16)]
                base = idx16 * DIM
                obase = iota32 + i * (16 * DIM)
                for dg in range(0, DIM, 16):
                    vals = [plsc.load_gather(table_v, [base + (dg + k)])
                            for k in range(16)]
                    for k in range(16):
                        plsc.store_scatter(out_buf, [obase + (dg + k)], vals[k])

        pltpu.async_copy(idx_src(0), idx_bufs[0], isems[0])
        pltpu.async_copy(idx_src(1), idx_bufs[1], isems[1])

        def body(g, carry):
            for p in (0, 1):
                @pl.when(lax.rem(g, 2) == p)
                def _():
                    pltpu.make_async_copy(idx_src(g), idx_bufs[p], isems[p]).wait()

                    @pl.when(g >= 2)
                    def _():
                        pltpu.make_async_copy(
                            out_bufs[p], out_dst(g - 2), osems[p]).wait()

                    compute(idx_bufs[p], out_bufs[p])
                    pltpu.async_copy(out_bufs[p], out_dst(g), osems[p])

                    @pl.when(g + 2 < nchunk)
                    def _():
                        pltpu.async_copy(idx_src(g + 2), idx_bufs[p], isems[p])
            return carry

        lax.fori_loop(0, nchunk, body, 0)

        pltpu.make_async_copy(
            out_bufs[(nchunk - 2) % 2], out_dst(nchunk - 2),
            osems[(nchunk - 2) % 2]).wait()
        pltpu.make_async_copy(
            out_bufs[(nchunk - 1) % 2], out_dst(nchunk - 1),
            osems[(nchunk - 1) % 2]).wait()

    return gather_kernel


def kernel(type_indices, embedding_table):
    idx = type_indices.reshape(N).astype(jnp.int32)
    table = embedding_table.reshape(ROWS * DIM)
    out = _build()(idx, table)
    return out.reshape(BATCH, HIST, DIM)


# speedup vs baseline: 2.1628x; 2.1628x over previous
import functools

import jax
import jax.numpy as jnp
from jax import lax
from jax.experimental import pallas as pl
from jax.experimental.pallas import tpu as pltpu
from jax.experimental.pallas import tpu_sc as plsc

BATCH = 16384
HIST = 200
DIM = 32
ROWS = 33
N = BATCH * HIST


@functools.cache
def _build():
    info = plsc.get_sparse_core_info()
    nw = info.num_cores * info.num_subcores
    per_w = N // nw
    chunk = 1280
    nchunk = per_w // chunk

    mesh = plsc.VectorSubcoreMesh(core_axis_name="c", subcore_axis_name="s")

    @functools.partial(
        pl.kernel,
        mesh=mesh,
        compiler_params=pltpu.CompilerParams(
            use_tc_tiling_on_sc=False, needs_layout_passes=False),
        out_type=jax.ShapeDtypeStruct((N * DIM,), jnp.float32),
        scratch_types=[
            pltpu.VMEM((ROWS * DIM,), jnp.float32),
            pltpu.VMEM((chunk,), jnp.int32),
            pltpu.VMEM((chunk,), jnp.int32),
            pltpu.VMEM((chunk * DIM,), jnp.float32),
            pltpu.VMEM((chunk * DIM,), jnp.float32),
            pltpu.SemaphoreType.DMA,
            pltpu.SemaphoreType.DMA,
            pltpu.SemaphoreType.DMA,
            pltpu.SemaphoreType.DMA,
        ],
    )
    def gather_kernel(idx_hbm, table_hbm, out_hbm, table_v,
                      idx0, idx1, outv0, outv1, si0, si1, so0, so1):
        wid = lax.axis_index("s") * info.num_cores + lax.axis_index("c")
        base_w = wid * per_w
        idx_bufs = (idx0, idx1)
        out_bufs = (outv0, outv1)
        isems = (si0, si1)
        osems = (so0, so1)

        pltpu.sync_copy(table_hbm, table_v)
        iota32 = lax.iota(jnp.int32, 16) * DIM

        def idx_src(g):
            return idx_hbm.at[pl.ds(base_w + g * chunk, chunk)]

        def out_dst(g):
            return out_hbm.at[pl.ds((base_w + g * chunk) * DIM, chunk * DIM)]

        iota16 = lax.iota(jnp.int32, 16)

        def compute(idx_buf, out_buf):
            def inner(i, carry):
                idx16 = idx_buf[pl.ds(i * 16, 16)]
                base = idx16 * DIM
                obase = iota32 + i * (16 * DIM)
                for tg in range(0, DIM, 16):
                    addrs = []
                    vals = []
                    for k in range(16):
                        dvec = jnp.bitwise_xor(iota16, tg + k)
                        vals.append(plsc.load_gather(table_v, [base + dvec]))
                        addrs.append(obase + dvec)
                    for k in range(16):
                        plsc.store_scatter(out_buf, [addrs[k]], vals[k])
                return carry
            lax.fori_loop(0, chunk // 16, inner, 0)

        pltpu.async_copy(idx_src(0), idx_bufs[0], isems[0])
        pltpu.async_copy(idx_src(1), idx_bufs[1], isems[1])

        def body(g, carry):
            for p in (0, 1):
                @pl.when(lax.rem(g, 2) == p)
                def _():
                    pltpu.make_async_copy(idx_src(g), idx_bufs[p], isems[p]).wait()

                    @pl.when(g >= 2)
                    def _():
                        pltpu.make_async_copy(
                            out_bufs[p], out_dst(g - 2), osems[p]).wait()

                    compute(idx_bufs[p], out_bufs[p])
                    pltpu.async_copy(out_bufs[p], out_dst(g), osems[p])

                    @pl.when(g + 2 < nchunk)
                    def _():
                        pltpu.async_copy(idx_src(g + 2), idx_bufs[p], isems[p])
            return carry

        lax.fori_loop(0, nchunk, body, 0)

        pltpu.make_async_copy(
            out_bufs[(nchunk - 2) % 2], out_dst(nchunk - 2),
            osems[(nchunk - 2) % 2]).wait()
        pltpu.make_async_copy(
            out_bufs[(nchunk - 1) % 2], out_dst(nchunk - 1),
            osems[(nchunk - 1) % 2]).wait()

    return gather_kernel


def kernel(type_indices, embedding_table):
    idx = type_indices.reshape(N).astype(jnp.int32)
    table = embedding_table.reshape(ROWS * DIM)
    out = _build()(idx, table)
    return out.reshape(BATCH, HIST, DIM)


# speedup vs baseline: 2.2685x; 1.0489x over previous
import functools

import jax
import jax.numpy as jnp
from jax import lax
from jax.experimental import pallas as pl
from jax.experimental.pallas import tpu as pltpu
from jax.experimental.pallas import tpu_sc as plsc

BATCH = 16384
HIST = 200
DIM = 32
ROWS = 33


@functools.cache
def _build():
    info = plsc.get_sparse_core_info()
    nw = info.num_cores * info.num_subcores
    rows_w = BATCH // nw
    R = 8
    nchunk = rows_w // R
    npos = R * HIST

    mesh = plsc.VectorSubcoreMesh(core_axis_name="c", subcore_axis_name="s")

    @functools.partial(
        pl.kernel,
        mesh=mesh,
        compiler_params=pltpu.CompilerParams(
            use_tc_tiling_on_sc=False, needs_layout_passes=False),
        out_type=jax.ShapeDtypeStruct((BATCH, HIST, DIM), jnp.float32),
        scratch_types=[
            pltpu.VMEM((ROWS * DIM,), jnp.float32),
            pltpu.VMEM((R, HIST), jnp.int32),
            pltpu.VMEM((R, HIST), jnp.int32),
            pltpu.VMEM((R, HIST, DIM), jnp.float32),
            pltpu.VMEM((R, HIST, DIM), jnp.float32),
            pltpu.SemaphoreType.DMA,
            pltpu.SemaphoreType.DMA,
            pltpu.SemaphoreType.DMA,
            pltpu.SemaphoreType.DMA,
        ],
    )
    def gather_kernel(idx_hbm, table_hbm, out_hbm, table_v,
                      idx0, idx1, outv0, outv1, si0, si1, so0, so1):
        wid = lax.axis_index("s") * info.num_cores + lax.axis_index("c")
        base_row = wid * rows_w
        idx_bufs = (idx0, idx1)
        out_bufs = (outv0, outv1)
        isems = (si0, si1)
        osems = (so0, so1)

        pltpu.sync_copy(table_hbm, table_v)
        iota16 = lax.iota(jnp.int32, 16)

        def idx_src(g):
            return idx_hbm.at[pl.ds(base_row + g * R, R)]

        def out_dst(g):
            return out_hbm.at[pl.ds(base_row + g * R, R)]

        def compute(idx_buf, out_buf):
            def inner(i, carry):
                row16, hist16 = carry
                idx16 = plsc.load_gather(idx_buf, [row16, hist16])
                base = idx16 * DIM
                for tg in range(0, DIM, 16):
                    dvecs = []
                    vals = []
                    for k in range(16):
                        dvec = jnp.bitwise_xor(iota16, tg + k)
                        vals.append(plsc.load_gather(table_v, [base + dvec]))
                        dvecs.append(dvec)
                    for k in range(16):
                        plsc.store_scatter(
                            out_buf, [row16, hist16, dvecs[k]], vals[k])
                hist_n = hist16 + 16
                wrap = hist_n >= HIST
                hist_n = jnp.where(wrap, hist_n - HIST, hist_n)
                row_n = row16 + wrap.astype(jnp.int32)
                return (row_n, hist_n)
            lax.fori_loop(0, npos // 16, inner,
                          (jnp.zeros((16,), jnp.int32), iota16))

        pltpu.async_copy(idx_src(0), idx_bufs[0], isems[0])
        pltpu.async_copy(idx_src(1), idx_bufs[1], isems[1])

        def body(g, carry):
            for p in (0, 1):
                @pl.when(lax.rem(g, 2) == p)
                def _():
                    pltpu.make_async_copy(idx_src(g), idx_bufs[p], isems[p]).wait()

                    @pl.when(g >= 2)
                    def _():
                        pltpu.make_async_copy(
                            out_bufs[p], out_dst(g - 2), osems[p]).wait()

                    compute(idx_bufs[p], out_bufs[p])
                    pltpu.async_copy(out_bufs[p], out_dst(g), osems[p])

                    @pl.when(g + 2 < nchunk)
                    def _():
                        pltpu.async_copy(idx_src(g + 2), idx_bufs[p], isems[p])
            return carry

        lax.fori_loop(0, nchunk, body, 0)

        pltpu.make_async_copy(
            out_bufs[(nchunk - 2) % 2], out_dst(nchunk - 2),
            osems[(nchunk - 2) % 2]).wait()
        pltpu.make_async_copy(
            out_bufs[(nchunk - 1) % 2], out_dst(nchunk - 1),
            osems[(nchunk - 1) % 2]).wait()

    return gather_kernel


def kernel(type_indices, embedding_table):
    idx = type_indices.astype(jnp.int32)
    table = embedding_table.reshape(ROWS * DIM)
    return _build()(idx, table)


# speedup vs baseline: 4.0738x; 1.7958x over previous
import functools

import jax
import jax.numpy as jnp
from jax import lax
from jax.experimental import pallas as pl
from jax.experimental.pallas import tpu as pltpu
from jax.experimental.pallas import tpu_sc as plsc

BATCH = 16384
HIST = 200
DIM = 32
ROWS = 33
TAB2 = 1064


@functools.cache
def _build():
    info = plsc.get_sparse_core_info()
    nw = info.num_cores * info.num_subcores
    nbt = BATCH // 128
    bt_w = nbt // nw
    b_w = bt_w * 128
    npiece = HIST * (DIM // 8)

    mesh = plsc.VectorSubcoreMesh(core_axis_name="c", subcore_axis_name="s")

    @functools.partial(
        pl.kernel,
        mesh=mesh,
        compiler_params=pltpu.CompilerParams(
            use_tc_tiling_on_sc=False, needs_layout_passes=False),
        out_type=jax.ShapeDtypeStruct((HIST, DIM // 8, nbt, 8, 128),
                                      jnp.float32),
        scratch_types=[
            pltpu.VMEM((TAB2 + ROWS * DIM,), jnp.float32),
            pltpu.VMEM((HIST, b_w), jnp.int32),
            pltpu.VMEM((bt_w, 8, 128), jnp.float32),
            pltpu.VMEM((bt_w, 8, 128), jnp.float32),
            pltpu.SemaphoreType.DMA,
            pltpu.SemaphoreType.DMA,
        ],
    )
    def gather_kernel(idx_hbm, table_hbm, out_hbm, table_v,
                      idx_v, outv0, outv1, so0, so1):
        wid = lax.axis_index("s") * info.num_cores + lax.axis_index("c")
        bt0 = wid * bt_w
        out_bufs = (outv0, outv1)
        osems = (so0, so1)

        pltpu.sync_copy(table_hbm, table_v.at[pl.ds(0, ROWS * DIM)])
        pltpu.sync_copy(table_hbm, table_v.at[pl.ds(TAB2, ROWS * DIM)])
        pltpu.sync_copy(idx_hbm.at[:, pl.ds(bt0 * 128, b_w)], idx_v)

        iota16 = lax.iota(jnp.int32, 16)
        hi8 = (iota16 >= 8).astype(jnp.int32) * TAB2
        dvecs = [(jnp.bitwise_xor(iota16, t) & 7) for t in range(8)]
        gvecs = [dvecs[t] + hi8 for t in range(8)]

        def out_dst(u):
            h = u // (DIM // 8)
            dt = lax.rem(u, DIM // 8)
            return out_hbm.at[h, dt, pl.ds(bt0, bt_w)]

        def compute(u, out_buf):
            h = u // (DIM // 8)
            dt = lax.rem(u, DIM // 8)

            def grp(rg, carry):
                r = rg // 8
                g = lax.rem(rg, 8)
                idx16 = idx_v[h, pl.ds(rg * 16, 16)]
                base = idx16 * DIM + dt * 8
                btv = jnp.broadcast_to(r, (16,))
                blv = iota16 + g * 16
                for t in range(8):
                    val = plsc.load_gather(table_v, [base + gvecs[t]])
                    plsc.store_scatter(
                        out_buf, [btv, dvecs[t], blv], val)
                return carry
            lax.fori_loop(0, bt_w * 8, grp, 0)

        def body(u, carry):
            for p in (0, 1):
                @pl.when(lax.rem(u, 2) == p)
                def _():
                    @pl.when(u >= 2)
                    def _():
                        pltpu.make_async_copy(
                            out_bufs[p], out_dst(u - 2), osems[p]).wait()
                    compute(u, out_bufs[p])
                    pltpu.async_copy(out_bufs[p], out_dst(u), osems[p])
            return carry

        lax.fori_loop(0, npiece, body, 0)

        pltpu.make_async_copy(
            out_bufs[(npiece - 2) % 2], out_dst(npiece - 2),
            osems[(npiece - 2) % 2]).wait()
        pltpu.make_async_copy(
            out_bufs[(npiece - 1) % 2], out_dst(npiece - 1),
            osems[(npiece - 1) % 2]).wait()

    return gather_kernel


def kernel(type_indices, embedding_table):
    idx_t = jnp.transpose(type_indices.astype(jnp.int32))
    table = embedding_table.reshape(ROWS * DIM)
    p = _build()(idx_t, table)
    return p.transpose(2, 4, 0, 1, 3).reshape(BATCH, HIST, DIM)


# speedup vs baseline: 9.6582x; 2.3708x over previous
import functools

import jax
import jax.numpy as jnp
from jax import lax
from jax.experimental import pallas as pl
from jax.experimental.pallas import tpu as pltpu
from jax.experimental.pallas import tpu_sc as plsc

BATCH = 16384
HIST = 200
DIM = 32
ROWS = 33
TAB2 = 1064


@functools.cache
def _build():
    info = plsc.get_sparse_core_info()
    nw = info.num_cores * info.num_subcores
    nbt = BATCH // 128
    bt_w = nbt // nw
    b_w = bt_w * 128
    npiece = HIST * (DIM // 8)

    mesh = plsc.VectorSubcoreMesh(core_axis_name="c", subcore_axis_name="s")

    @functools.partial(
        pl.kernel,
        mesh=mesh,
        compiler_params=pltpu.CompilerParams(
            use_tc_tiling_on_sc=False, needs_layout_passes=False),
        out_type=jax.ShapeDtypeStruct((HIST * (DIM // 8), nbt * 8 * 128),
                                      jnp.float32),
        scratch_types=[
            pltpu.VMEM((TAB2 + ROWS * DIM,), jnp.float32),
            pltpu.VMEM((HIST, b_w), jnp.int32),
            pltpu.VMEM((bt_w * 8 * 128,), jnp.float32),
            pltpu.VMEM((bt_w * 8 * 128,), jnp.float32),
            pltpu.SemaphoreType.DMA,
            pltpu.SemaphoreType.DMA,
        ],
    )
    def gather_kernel(idx_hbm, table_hbm, out_hbm, table_v,
                      idx_v, outv0, outv1, so0, so1):
        wid = lax.axis_index("s") * info.num_cores + lax.axis_index("c")
        bt0 = wid * bt_w
        out_bufs = (outv0, outv1)
        osems = (so0, so1)

        pltpu.sync_copy(table_hbm, table_v.at[pl.ds(0, ROWS * DIM)])
        pltpu.sync_copy(table_hbm, table_v.at[pl.ds(TAB2, ROWS * DIM)])
        pltpu.sync_copy(idx_hbm.at[:, pl.ds(bt0 * 128, b_w)], idx_v)

        iota16 = lax.iota(jnp.int32, 16)
        hi8 = (iota16 >= 8).astype(jnp.int32) * TAB2
        dvecs = [(jnp.bitwise_xor(iota16, t) & 7) for t in range(8)]
        gvecs = [dvecs[t] + hi8 for t in range(8)]
        svecs = [dvecs[t] * 128 + iota16 for t in range(8)]

        def out_dst(u):
            return out_hbm.at[u, pl.ds(bt0 * 1024, bt_w * 1024)]

        def compute(u, out_buf):
            h = u // (DIM // 8)
            dt8 = lax.rem(u, DIM // 8) * 8
            pvecs = [gvecs[t] + dt8 for t in range(8)]

            def grp(i, carry):
                for sub in (0, 1):
                    rg = i * 2 + sub
                    r = rg // 8
                    g = lax.rem(rg, 8)
                    idx16 = idx_v[h, pl.ds(rg * 16, 16)]
                    base = idx16 * DIM
                    obase = r * 1024 + g * 16
                    vals = [plsc.load_gather(table_v, [base + pvecs[t]])
                            for t in range(8)]
                    for t in range(8):
                        plsc.store_scatter(
                            out_buf, [svecs[t] + obase], vals[t])
                return carry
            lax.fori_loop(0, bt_w * 4, grp, 0)

        nb = 2

        def body(u, carry):
            for p in range(nb):
                @pl.when(lax.rem(u, nb) == p)
                def _():
                    @pl.when(u >= nb)
                    def _():
                        pltpu.make_async_copy(
                            out_bufs[p], out_dst(u - nb), osems[p]).wait()
                    compute(u, out_bufs[p])
                    pltpu.async_copy(out_bufs[p], out_dst(u), osems[p])
            return carry

        lax.fori_loop(0, npiece, body, 0)

        for k in range(nb):
            u = npiece - nb + k
            pltpu.make_async_copy(
                out_bufs[u % nb], out_dst(u), osems[u % nb]).wait()

    return gather_kernel


def kernel(type_indices, embedding_table):
    idx_t = jnp.transpose(type_indices.astype(jnp.int32))
    table = embedding_table.reshape(ROWS * DIM)
    p = _build()(idx_t, table)
    p = p.reshape(HIST, DIM // 8, BATCH // 128, 8, 128)
    return p.transpose(2, 4, 0, 1, 3).reshape(BATCH, HIST, DIM)
